# baseline (speedup 1.0000x reference)
import jax
import jax.numpy as jnp
from jax.experimental import pallas as pl
from jax.experimental.pallas import tpu as pltpu

_N = 4096
_D = 64
_BME = 256
_NBE = _N // _BME
_BMP = 512
_NBP = _N // _BMP
_PH = [_NBE, _NBE + _NBP, _NBE + 2 * _NBP, _NBE + 3 * _NBP, _NBE + 4 * _NBP]
_BM_ID = 256


def _l2n(x):
    n = jnp.sqrt(jnp.sum(x * x, axis=1, keepdims=True))
    return x / jnp.maximum(n, 1e-12)


def _lrelu(x):
    return jnp.where(x >= 0, x, 0.01 * x)


def _dot(a, b):
    return jnp.dot(a, b.astype(jnp.float32),
                   preferred_element_type=jnp.float32)


def _row_spec(bm, w):
    return pl.BlockSpec((bm, w), lambda i: (i, 0))


def _full_spec(h, w):
    return pl.BlockSpec((h, w), lambda i: (0, 0))


_PARAMS = pltpu.CompilerParams(dimension_semantics=("arbitrary",))


def _id_body(ui0, ui1, iu0, iu1, iemb, uemb, wcat, ue, ie, ou, oi):
    wc = wcat[...]
    ws = wc[0:64] + wc[64:128] + wc[128:192] + wc[192:256]
    eu = _dot(iemb[...], ws) * 0.5
    ei = _dot(uemb[...], ws) * 0.5
    um = _dot(ui0[...] + ui1[...], eu)
    im = _dot(iu0[...] + iu1[...], ei)
    ou[...] = ue[...] + 0.36 * _l2n(um)
    oi[...] = ie[...] + 0.36 * _l2n(im)


def _mega_body(f0, f1, ui, iu, ug0, ig0,
               w10, b10, w20, b20, w11, b11, w21, b21,
               ufin, ifin,
               r0, r1, r2, r3, i1s):
    i = pl.program_id(0)

    @pl.when(i < _PH[0])
    def _enc():
        k = i
        rows = pl.ds(k * _BME, _BME)
        h0 = _lrelu(_dot(f0[...], w10[...]) + b10[...])
        r0[rows, 0:64] = _lrelu(_dot(h0, w20[...]) + b20[...]).astype(jnp.bfloat16)
        h1 = _lrelu(_dot(f1[...], w11[...]) + b11[...])
        r0[rows, 64:128] = _lrelu(_dot(h1, w21[...]) + b21[...]).astype(jnp.bfloat16)
        r0[rows, 128:192] = ig0[rows, :].astype(jnp.bfloat16)

    @pl.when((i >= _PH[0]) & (i < _PH[1]))
    def _pass_a():
        k = i - _PH[0]
        rows = pl.ds(k * _BMP, _BMP)
        t = _dot(ui[...], r0[...])
        r1[rows, :] = t.astype(jnp.bfloat16)

    @pl.when((i >= _PH[1]) & (i < _PH[2]))
    def _pass_b():
        k = i - _PH[1]
        rows = pl.ds(k * _BMP, _BMP)
        t = _dot(iu[...], r1[...])
        r2[rows, 0:128] = t[:, 0:128].astype(jnp.bfloat16)
        s = t[:, 128:192]
        i1s[rows, :] = s.astype(jnp.bfloat16)
        s = s - jnp.max(s, axis=1, keepdims=True)
        e = jnp.exp(s)
        r2[rows, 128:192] = (e / jnp.sum(e, axis=1, keepdims=True)).astype(jnp.bfloat16)

    @pl.when((i >= _PH[2]) & (i < _PH[3]))
    def _pass_c():
        k = i - _PH[2]
        rows = pl.ds(k * _BMP, _BMP)
        t = _dot(ui[...], r2[...])
        r3[rows, :] = t.astype(jnp.bfloat16)
        u1 = r1[rows, 128:192].astype(jnp.float32)
        ufin[...] = (ug0[rows, :] + u1 + t[:, 128:192]) / 3.0 + \
            0.02 * (_l2n(t[:, 0:64]) + _l2n(t[:, 64:128]))

    @pl.when(i >= _PH[3])
    def _pass_d():
        k = i - _PH[3]
        rows = pl.ds(k * _BMP, _BMP)
        t = _dot(iu[...], r3[...])
        ifin[...] = (ig0[rows, :] + i1s[rows, :].astype(jnp.float32)
                     + t[:, 128:192]) / 3.0 + \
            0.02 * (_l2n(t[:, 0:64]) + _l2n(t[:, 64:128]))


def kernel(ui_graph, iu_graph, mm_ui_graph_0, mm_ui_graph_1, mm_iu_graph_0,
           mm_iu_graph_1, mm_feats_0, mm_feats_1,
           enc0_W1, enc0_b1, enc0_W2, enc0_b2,
           enc1_W1, enc1_b1, enc1_W2, enc1_b2,
           user_emb, item_emb, w_q, w_k, w_cat):
    del w_q, w_k
    f32 = jnp.float32
    k1 = enc0_W1.shape[1]
    k2 = enc1_W1.shape[0]
    k3 = enc1_W1.shape[1]

    n_blk_id = _N // _BM_ID
    ug0, ig0 = pl.pallas_call(
        _id_body,
        grid=(n_blk_id,),
        in_specs=[
            _row_spec(_BM_ID, _N), _row_spec(_BM_ID, _N),
            _row_spec(_BM_ID, _N), _row_spec(_BM_ID, _N),
            _full_spec(_N, _D), _full_spec(_N, _D),
            _full_spec(4 * _D, _D),
            _row_spec(_BM_ID, _D), _row_spec(_BM_ID, _D),
        ],
        out_specs=[_row_spec(_BM_ID, _D), _row_spec(_BM_ID, _D)],
        out_shape=[jax.ShapeDtypeStruct((_N, _D), f32)] * 2,
        compiler_params=pltpu.CompilerParams(
            dimension_semantics=("parallel",)),
    )(mm_ui_graph_0, mm_ui_graph_1, mm_iu_graph_0, mm_iu_graph_1,
      item_emb, user_emb, w_cat, user_emb, item_emb)

    w = 3 * _D

    def _clip(x, lo, hi):
        return jnp.minimum(jnp.maximum(x, lo), hi)

    f0_spec = pl.BlockSpec((_BME, _N), lambda i: (_clip(i, 0, _NBE - 1), 0))
    f1_spec = pl.BlockSpec((_BME, k2), lambda i: (_clip(i, 0, _NBE - 1), 0))
    ui_spec = pl.BlockSpec(
        (_BMP, _N),
        lambda i: (jnp.where(i < _PH[1],
                             _clip(i - _PH[0], 0, _NBP - 1),
                             _clip(i - _PH[2], 0, _NBP - 1)), 0))
    iu_spec = pl.BlockSpec(
        (_BMP, _N),
        lambda i: (jnp.where(i < _PH[2],
                             _clip(i - _PH[1], 0, _NBP - 1),
                             _clip(i - _PH[3], 0, _NBP - 1)), 0))
    ufin_spec = pl.BlockSpec((_BMP, _D),
                             lambda i: (_clip(i - _PH[2], 0, _NBP - 1), 0))
    ifin_spec = pl.BlockSpec((_BMP, _D),
                             lambda i: (_clip(i - _PH[3], 0, _NBP - 1), 0))

    u_final, i_final = pl.pallas_call(
        _mega_body,
        grid=(_PH[4],),
        in_specs=[
            f0_spec, f1_spec, ui_spec, iu_spec,
            _full_spec(_N, _D), _full_spec(_N, _D),
            _full_spec(_N, k1), _full_spec(1, k1),
            _full_spec(k1, _D), _full_spec(1, _D),
            _full_spec(k2, k3), _full_spec(1, k3),
            _full_spec(k3, _D), _full_spec(1, _D),
        ],
        out_specs=[ufin_spec, ifin_spec],
        out_shape=[jax.ShapeDtypeStruct((_N, _D), f32)] * 2,
        scratch_shapes=[
            pltpu.VMEM((_N, w), jnp.bfloat16),
            pltpu.VMEM((_N, w), jnp.bfloat16),
            pltpu.VMEM((_N, w), jnp.bfloat16),
            pltpu.VMEM((_N, w), jnp.bfloat16),
            pltpu.VMEM((_N, _D), jnp.bfloat16),
        ],
        compiler_params=_PARAMS,
    )(mm_feats_0, mm_feats_1, ui_graph, iu_graph, ug0, ig0,
      enc0_W1, enc0_b1.reshape(1, -1), enc0_W2, enc0_b2.reshape(1, -1),
      enc1_W1, enc1_b1.reshape(1, -1), enc1_W2, enc1_b2.reshape(1, -1))

    return u_final, i_final


# speedup vs baseline: 1.0369x; 1.0369x over previous
import jax
import jax.numpy as jnp
from jax.experimental import pallas as pl
from jax.experimental.pallas import tpu as pltpu

_N = 4096
_D = 64
_BM1 = 256
_NB1 = _N // _BM1
_BMP = 512
_NBP = _N // _BMP
_BF = jnp.bfloat16


def _l2n(x):
    n = jnp.sqrt(jnp.sum(x * x, axis=1, keepdims=True))
    return x / jnp.maximum(n, 1e-12)


def _lrelu(x):
    return jnp.where(x >= 0, x, 0.01 * x)


def _dot(a, b):
    return jnp.dot(a.astype(_BF), b.astype(_BF),
                   preferred_element_type=jnp.float32)


def _row_spec(bm, w):
    return pl.BlockSpec((bm, w), lambda i: (i, 0))


def _full_spec(h, w):
    return pl.BlockSpec((h, w), lambda i: (0, 0))


_PARAMS = pltpu.CompilerParams(dimension_semantics=("arbitrary",))


def _id_enc_body(ui0, ui1, iu0, iu1, f0, f1, iemb, uemb, wcat, ue, ie,
                 w10, b10, w20, b20, w11, b11, w21, b21,
                 ou, oi, oif):
    wc = wcat[...]
    ws = wc[0:64] + wc[64:128] + wc[128:192] + wc[192:256]
    eu = _dot(iemb[...], ws) * 0.5
    ei = _dot(uemb[...], ws) * 0.5
    um = _dot(ui0[...] + ui1[...], eu)
    im = _dot(iu0[...] + iu1[...], ei)
    ou[...] = ue[...] + 0.36 * _l2n(um)
    oi[...] = ie[...] + 0.36 * _l2n(im)
    h0 = _lrelu(_dot(f0[...], w10[...]) + b10[...])
    oif[:, 0:64] = _lrelu(_dot(h0, w20[...]) + b20[...]).astype(_BF)
    h1 = _lrelu(_dot(f1[...], w11[...]) + b11[...])
    oif[:, 64:128] = _lrelu(_dot(h1, w21[...]) + b21[...]).astype(_BF)


def _mega_body(ui, iu, ug0, ig0, if01, ufin, ifin, r0, r1, r2, r3, i1s):
    i = pl.program_id(0)

    @pl.when(i == 0)
    def _fill_r0():
        r0[:, 0:128] = if01[...]
        r0[:, 128:192] = ig0[...].astype(_BF)

    @pl.when(i < _NBP)
    def _pass_a():
        rows = pl.ds(i * _BMP, _BMP)
        t = _dot(ui[...], r0[...])
        r1[rows, :] = t.astype(_BF)

    @pl.when((i >= _NBP) & (i < 2 * _NBP))
    def _pass_b():
        rows = pl.ds((i - _NBP) * _BMP, _BMP)
        t = _dot(iu[...], r1[...])
        r2[rows, 0:128] = t[:, 0:128].astype(_BF)
        s = t[:, 128:192]
        i1s[rows, :] = s.astype(_BF)
        s = s - jnp.max(s, axis=1, keepdims=True)
        e = jnp.exp(s)
        r2[rows, 128:192] = (e / jnp.sum(e, axis=1, keepdims=True)).astype(_BF)

    @pl.when((i >= 2 * _NBP) & (i < 3 * _NBP))
    def _pass_c():
        rows = pl.ds((i - 2 * _NBP) * _BMP, _BMP)
        t = _dot(ui[...], r2[...])
        r3[rows, :] = t.astype(_BF)
        u1 = r1[rows, 128:192].astype(jnp.float32)
        ufin[...] = (ug0[rows, :] + u1 + t[:, 128:192]) / 3.0 + \
            0.02 * (_l2n(t[:, 0:64]) + _l2n(t[:, 64:128]))

    @pl.when(i >= 3 * _NBP)
    def _pass_d():
        rows = pl.ds((i - 3 * _NBP) * _BMP, _BMP)
        t = _dot(iu[...], r3[...])
        ifin[...] = (ig0[rows, :] + i1s[rows, :].astype(jnp.float32)
                     + t[:, 128:192]) / 3.0 + \
            0.02 * (_l2n(t[:, 0:64]) + _l2n(t[:, 64:128]))


def kernel(ui_graph, iu_graph, mm_ui_graph_0, mm_ui_graph_1, mm_iu_graph_0,
           mm_iu_graph_1, mm_feats_0, mm_feats_1,
           enc0_W1, enc0_b1, enc0_W2, enc0_b2,
           enc1_W1, enc1_b1, enc1_W2, enc1_b2,
           user_emb, item_emb, w_q, w_k, w_cat):
    del w_q, w_k
    f32 = jnp.float32
    k1 = enc0_W1.shape[1]
    k2 = enc1_W1.shape[0]
    k3 = enc1_W1.shape[1]

    ug0, ig0, if01 = pl.pallas_call(
        _id_enc_body,
        grid=(_NB1,),
        in_specs=[
            _row_spec(_BM1, _N), _row_spec(_BM1, _N),
            _row_spec(_BM1, _N), _row_spec(_BM1, _N),
            _row_spec(_BM1, _N), _row_spec(_BM1, k2),
            _full_spec(_N, _D), _full_spec(_N, _D), _full_spec(4 * _D, _D),
            _row_spec(_BM1, _D), _row_spec(_BM1, _D),
            _full_spec(_N, k1), _full_spec(1, k1),
            _full_spec(k1, _D), _full_spec(1, _D),
            _full_spec(k2, k3), _full_spec(1, k3),
            _full_spec(k3, _D), _full_spec(1, _D),
        ],
        out_specs=[_row_spec(_BM1, _D), _row_spec(_BM1, _D),
                   _row_spec(_BM1, 2 * _D)],
        out_shape=[jax.ShapeDtypeStruct((_N, _D), f32),
                   jax.ShapeDtypeStruct((_N, _D), f32),
                   jax.ShapeDtypeStruct((_N, 2 * _D), _BF)],
        compiler_params=_PARAMS,
    )(mm_ui_graph_0, mm_ui_graph_1, mm_iu_graph_0, mm_iu_graph_1,
      mm_feats_0, mm_feats_1, item_emb, user_emb, w_cat,
      user_emb, item_emb,
      enc0_W1, enc0_b1.reshape(1, -1), enc0_W2, enc0_b2.reshape(1, -1),
      enc1_W1, enc1_b1.reshape(1, -1), enc1_W2, enc1_b2.reshape(1, -1))

    w = 3 * _D
    p = _NBP

    def _clip(x, lo, hi):
        return jnp.minimum(jnp.maximum(x, lo), hi)

    ui_spec = pl.BlockSpec(
        (_BMP, _N),
        lambda i: (jnp.where(i < 2 * p, _clip(i, 0, p - 1),
                             _clip(i - 2 * p, 0, p - 1)), 0))
    iu_spec = pl.BlockSpec(
        (_BMP, _N),
        lambda i: (jnp.where(i < 3 * p, _clip(i - p, 0, p - 1),
                             _clip(i - 3 * p, 0, p - 1)), 0))
    ufin_spec = pl.BlockSpec((_BMP, _D),
                             lambda i: (_clip(i - 2 * p, 0, p - 1), 0))
    ifin_spec = pl.BlockSpec((_BMP, _D),
                             lambda i: (_clip(i - 3 * p, 0, p - 1), 0))

    u_final, i_final = pl.pallas_call(
        _mega_body,
        grid=(4 * p,),
        in_specs=[ui_spec, iu_spec,
                  _full_spec(_N, _D), _full_spec(_N, _D),
                  _full_spec(_N, 2 * _D)],
        out_specs=[ufin_spec, ifin_spec],
        out_shape=[jax.ShapeDtypeStruct((_N, _D), f32)] * 2,
        scratch_shapes=[
            pltpu.VMEM((_N, w), _BF),
            pltpu.VMEM((_N, w), _BF),
            pltpu.VMEM((_N, w), _BF),
            pltpu.VMEM((_N, w), _BF),
            pltpu.VMEM((_N, _D), _BF),
        ],
        compiler_params=_PARAMS,
    )(ui_graph, iu_graph, ug0, ig0, if01)

    return u_final, i_final
